# baseline (speedup 1.0000x reference)
import functools

import jax
import jax.numpy as jnp
from jax.experimental import pallas as pl


def _layer_side_body(a_ref, hot_ref, hs_ref, wst_ref, wnt_ref, o_ref):
    neigh = jnp.dot(a_ref[...], hot_ref[...], preferred_element_type=jnp.float32)
    x = (
        jnp.dot(hs_ref[...], wst_ref[...], preferred_element_type=jnp.float32)
        + jnp.dot(neigh, wnt_ref[...], preferred_element_type=jnp.float32)
    )
    x = jnp.maximum(x, 0.0)
    n = jnp.sqrt(jnp.sum(x * x, axis=1, keepdims=True))
    o_ref[...] = x / jnp.maximum(n, 1e-12)


@functools.partial(jax.jit, static_argnames=("bm",))
def _layer_side(A, h_other, h_self, W, bm=200):
    M, K = A.shape
    D = h_self.shape[1]
    wst = W[:, :D].T
    wnt = W[:, D:].T
    return pl.pallas_call(
        _layer_side_body,
        grid=(M // bm,),
        in_specs=[
            pl.BlockSpec((bm, K), lambda i: (i, 0)),
            pl.BlockSpec((K, D), lambda i: (0, 0)),
            pl.BlockSpec((bm, D), lambda i: (i, 0)),
            pl.BlockSpec((D, D), lambda i: (0, 0)),
            pl.BlockSpec((D, D), lambda i: (0, 0)),
        ],
        out_specs=pl.BlockSpec((bm, D), lambda i: (i, 0)),
        out_shape=jax.ShapeDtypeStruct((M, D), jnp.float32),
    )(A, h_other, h_self, wst, wnt)


def kernel(adj_u2i, adj_i2u, user_emb, item_emb, W_user0, W_user1, W_item0, W_item1):
    h_u, h_i = user_emb, item_emb
    for Wu, Wi in ((W_user0, W_item0), (W_user1, W_item1)):
        h_u_new = _layer_side(adj_u2i, h_i, h_u, Wu)
        h_i_new = _layer_side(adj_i2u, h_u, h_i, Wi)
        h_u, h_i = h_u_new, h_i_new
    return (h_u, h_i)


# speedup vs baseline: 1.0297x; 1.0297x over previous
import functools

import jax
import jax.numpy as jnp
from jax.experimental import pallas as pl


def _layer_side_body(a_ref, hot_ref, hs_ref, wst_ref, wnt_ref, o_ref):
    neigh = jnp.dot(
        a_ref[...].astype(jnp.bfloat16),
        hot_ref[...].astype(jnp.bfloat16),
        preferred_element_type=jnp.float32,
    )
    x = (
        jnp.dot(hs_ref[...], wst_ref[...], preferred_element_type=jnp.float32)
        + jnp.dot(neigh, wnt_ref[...], preferred_element_type=jnp.float32)
    )
    x = jnp.maximum(x, 0.0)
    n = jnp.sqrt(jnp.sum(x * x, axis=1, keepdims=True))
    o_ref[...] = x / jnp.maximum(n, 1e-12)


@functools.partial(jax.jit, static_argnames=("bm",))
def _layer_side(A, h_other, h_self, W, bm=400):
    M, K = A.shape
    D = h_self.shape[1]
    wst = W[:, :D].T
    wnt = W[:, D:].T
    return pl.pallas_call(
        _layer_side_body,
        grid=(M // bm,),
        in_specs=[
            pl.BlockSpec((bm, K), lambda i: (i, 0)),
            pl.BlockSpec((K, D), lambda i: (0, 0)),
            pl.BlockSpec((bm, D), lambda i: (i, 0)),
            pl.BlockSpec((D, D), lambda i: (0, 0)),
            pl.BlockSpec((D, D), lambda i: (0, 0)),
        ],
        out_specs=pl.BlockSpec((bm, D), lambda i: (i, 0)),
        out_shape=jax.ShapeDtypeStruct((M, D), jnp.float32),
    )(A, h_other, h_self, wst, wnt)


def kernel(adj_u2i, adj_i2u, user_emb, item_emb, W_user0, W_user1, W_item0, W_item1):
    h_u, h_i = user_emb, item_emb
    for Wu, Wi in ((W_user0, W_item0), (W_user1, W_item1)):
        h_u_new = _layer_side(adj_u2i, h_i, h_u, Wu)
        h_i_new = _layer_side(adj_i2u, h_u, h_i, Wi)
        h_u, h_i = h_u_new, h_i_new
    return (h_u, h_i)


# speedup vs baseline: 1.0530x; 1.0226x over previous
import functools

import jax
import jax.numpy as jnp
from jax.experimental import pallas as pl


def _layer_side_body(a0_ref, a1_ref, hot_ref, hs_ref, wst_ref, wnt_ref, o_ref):
    hot = hot_ref[...].astype(jnp.bfloat16)
    neigh = jnp.concatenate(
        [
            jnp.dot(a0_ref[...].astype(jnp.bfloat16), hot,
                    preferred_element_type=jnp.float32),
            jnp.dot(a1_ref[...].astype(jnp.bfloat16), hot,
                    preferred_element_type=jnp.float32),
        ],
        axis=0,
    )
    x = (
        jnp.dot(hs_ref[...], wst_ref[...], preferred_element_type=jnp.float32)
        + jnp.dot(neigh, wnt_ref[...], preferred_element_type=jnp.float32)
    )
    x = jnp.maximum(x, 0.0)
    n = jnp.sqrt(jnp.sum(x * x, axis=1, keepdims=True))
    o_ref[...] = x / jnp.maximum(n, 1e-12)


@functools.partial(jax.jit, static_argnames=("bm",))
def _layer_side(A, h_other, h_self, W, bm=400):
    M, K = A.shape
    D = h_self.shape[1]
    wst = W[:, :D].T
    wnt = W[:, D:].T
    return pl.pallas_call(
        _layer_side_body,
        grid=(M // bm,),
        in_specs=[
            pl.BlockSpec((bm // 2, K), lambda i: (2 * i, 0)),
            pl.BlockSpec((bm // 2, K), lambda i: (2 * i + 1, 0)),
            pl.BlockSpec((K, D), lambda i: (0, 0)),
            pl.BlockSpec((bm, D), lambda i: (i, 0)),
            pl.BlockSpec((D, D), lambda i: (0, 0)),
            pl.BlockSpec((D, D), lambda i: (0, 0)),
        ],
        out_specs=pl.BlockSpec((bm, D), lambda i: (i, 0)),
        out_shape=jax.ShapeDtypeStruct((M, D), jnp.float32),
    )(A, A, h_other, h_self, wst, wnt)


def kernel(adj_u2i, adj_i2u, user_emb, item_emb, W_user0, W_user1, W_item0, W_item1):
    h_u, h_i = user_emb, item_emb
    for Wu, Wi in ((W_user0, W_item0), (W_user1, W_item1)):
        h_u_new = _layer_side(adj_u2i, h_i, h_u, Wu)
        h_i_new = _layer_side(adj_i2u, h_u, h_i, Wi)
        h_u, h_i = h_u_new, h_i_new
    return (h_u, h_i)
